# Initial kernel scaffold; baseline (speedup 1.0000x reference)
import functools

import jax
import jax.numpy as jnp
from jax import lax
from jax.experimental import pallas as pl
from jax.experimental.pallas import tpu as pltpu
from jax.experimental.pallas import tpu_sc as plsc

N = 100000
C = 256
G = 64
H = 16
NW = 32
RPW = N // NW
CHUNK = 125
NCHUNK = RPW // CHUNK
BPAD = 144

_mesh = plsc.VectorSubcoreMesh(core_axis_name="c", subcore_axis_name="s")

f32 = jnp.float32
i32 = jnp.int32


@functools.partial(
    pl.kernel,
    out_type=(
        jax.ShapeDtypeStruct((NW, G * C), f32),
        jax.ShapeDtypeStruct((NW, G * 16), f32),
    ),
    mesh=_mesh,
    scratch_types=[
        pltpu.VMEM((CHUNK * C,), f32),
        pltpu.VMEM((BPAD,), i32),
        pltpu.VMEM((G * C,), f32),
        pltpu.VMEM((G * 16,), f32),
    ],
)
def _segsum(x_hbm, batch_hbm, sums_out, cnt_out, xbuf, bbuf, accs, accc):
    wid = lax.axis_index("s") * 2 + lax.axis_index("c")
    zeros = jnp.zeros((16,), f32)
    ones = jnp.ones((16,), f32)

    def _zs(k, carry):
        accs[pl.ds(k * 16, 16)] = zeros
        return carry

    lax.fori_loop(0, G * C // 16, _zs, 0)

    def _zc(k, carry):
        accc[pl.ds(k * 16, 16)] = zeros
        return carry

    lax.fori_loop(0, G, _zc, 0)

    row0 = wid * RPW

    def chunk_body(cix, carry):
        g0 = row0 + cix * CHUNK
        x0 = pl.multiple_of(g0 * C, 8)
        pltpu.sync_copy(x_hbm.at[pl.ds(x0, CHUNK * C)], xbuf)
        a0 = pl.multiple_of((g0 // 8) * 8, 8)
        off = g0 - a0
        pltpu.sync_copy(batch_hbm.at[pl.ds(a0, BPAD)], bbuf)

        def row_body(i, rc):
            seg = bbuf[off + i]
            bs = seg * C
            xb = i * C
            for j in range(16):
                xv = xbuf[pl.ds(xb + j * 16, 16)]
                plsc.addupdate(accs.at[pl.ds(bs + j * 16, 16)], xv)
            plsc.addupdate(accc.at[pl.ds(seg * 16, 16)], ones)
            return rc

        lax.fori_loop(0, CHUNK, row_body, 0)
        return carry

    lax.fori_loop(0, NCHUNK, chunk_body, 0)
    pltpu.sync_copy(accs, sums_out.at[wid])
    pltpu.sync_copy(accc, cnt_out.at[wid])


def _mlp_body(sp_ref, cp_ref, w1_ref, w2_ref, scale_ref, offs_ref):
    sums = jnp.sum(sp_ref[...], axis=0)
    cnt = jnp.sum(cp_ref[...], axis=0)
    counts = cnt[:, 0:1]
    inv = 1.0 / jnp.maximum(counts, 1.0)
    mean = sums * inv
    h = lax.dot_general(mean, w1_ref[...], (((1,), (1,)), ((), ())),
                        preferred_element_type=f32)
    h = jnp.maximum(h, 0.0)
    logits = lax.dot_general(h, w2_ref[...], (((1,), (1,)), ((), ())),
                             preferred_element_type=f32)
    scale_ref[...] = 1.0 / (1.0 + jnp.exp(-logits))
    ii = lax.broadcasted_iota(i32, (G, 128), 0)
    jj = lax.broadcasted_iota(i32, (G, 128), 1)
    contrib = jnp.where(ii < jj, counts, 0.0)
    offs = jnp.sum(contrib, axis=0, keepdims=True)
    offs_ref[...] = jnp.broadcast_to(offs, (8, 128)).astype(i32)


def _mlp(sums_p, cnt_p, W1, W2):
    return pl.pallas_call(
        _mlp_body,
        out_shape=(
            jax.ShapeDtypeStruct((G, C), f32),
            jax.ShapeDtypeStruct((8, 128), i32),
        ),
    )(sums_p, cnt_p, W1, W2)


@functools.partial(
    pl.kernel,
    out_type=jax.ShapeDtypeStruct((N * C,), f32),
    mesh=_mesh,
    scratch_types=[
        pltpu.VMEM((CHUNK * C,), f32),
        pltpu.VMEM((G * C,), f32),
        pltpu.VMEM((128,), i32),
    ],
)
def _scale_mul(x_hbm, scale_hbm, offs_hbm, out_hbm, buf, sc_v, offs_v):
    wid = lax.axis_index("s") * 2 + lax.axis_index("c")
    pltpu.sync_copy(scale_hbm, sc_v)
    pltpu.sync_copy(offs_hbm.at[0], offs_v)
    row0 = wid * RPW

    def chunk_body(cix, carry):
        r0 = row0 + cix * CHUNK
        x0 = pl.multiple_of(r0 * C, 8)
        pltpu.sync_copy(x_hbm.at[pl.ds(x0, CHUNK * C)], buf)

        def seg_body(g, sc):
            lo = jnp.maximum(offs_v[g], r0)
            hi = jnp.minimum(offs_v[g + 1], r0 + CHUNK)

            @pl.when(lo < hi)
            def _():
                sb = g * C
                svs = [sc_v[pl.ds(sb + j * 16, 16)] for j in range(16)]

                def row_body(i, rc):
                    b = i * C
                    for j in range(16):
                        s = pl.ds(b + j * 16, 16)
                        buf[s] = buf[s] * svs[j]
                    return rc

                lax.fori_loop(lo - r0, hi - r0, row_body, 0)

            return sc

        lax.fori_loop(0, G, seg_body, 0)
        pltpu.sync_copy(buf, out_hbm.at[pl.ds(x0, CHUNK * C)])
        return carry

    lax.fori_loop(0, NCHUNK, chunk_body, 0)


def kernel(x, batch, W1, W2):
    xf = x.reshape(-1)
    b32 = batch.astype(i32)
    bpad = jnp.concatenate([b32, jnp.zeros((96,), i32)])
    sums_p, cnt_p = _segsum(xf, bpad)
    scale, offs = _mlp(sums_p.reshape(NW, G, C), cnt_p.reshape(NW, G, 16), W1, W2)
    out = _scale_mul(xf, scale.reshape(-1), offs)
    return out.reshape(N, C)


# speedup vs baseline: 1.3029x; 1.3029x over previous
import functools

import jax
import jax.numpy as jnp
from jax import lax
from jax.experimental import pallas as pl
from jax.experimental.pallas import tpu as pltpu
from jax.experimental.pallas import tpu_sc as plsc

N = 100000
C = 256
G = 64
H = 16
NW = 32
RPW = N // NW
CHUNK = 125
NCHUNK = RPW // CHUNK
BPAD = 160

_mesh = plsc.VectorSubcoreMesh(core_axis_name="c", subcore_axis_name="s")

f32 = jnp.float32
i32 = jnp.int32


@functools.partial(
    pl.kernel,
    out_type=(
        jax.ShapeDtypeStruct((NW, G * C), f32),
        jax.ShapeDtypeStruct((NW, G * 16), f32),
    ),
    mesh=_mesh,
    scratch_types=[
        pltpu.VMEM((CHUNK * C,), f32),
        pltpu.VMEM((BPAD,), i32),
        pltpu.VMEM((G * C,), f32),
        pltpu.VMEM((G * 16,), f32),
    ],
)
def _segsum(x_hbm, batch_hbm, sums_out, cnt_out, xbuf, bbuf, accs, accc):
    wid = lax.axis_index("s") * 2 + lax.axis_index("c")
    zeros = jnp.zeros((16,), f32)
    ones = jnp.ones((16,), f32)

    def _zs(k, carry):
        accs[pl.ds(k * 16, 16)] = zeros
        return carry

    lax.fori_loop(0, G * C // 16, _zs, 0)

    def _zc(k, carry):
        accc[pl.ds(k * 16, 16)] = zeros
        return carry

    lax.fori_loop(0, G, _zc, 0)

    row0 = wid * RPW

    def chunk_body(cix, carry):
        g0 = row0 + cix * CHUNK
        x0 = pl.multiple_of(g0 * C, 8)
        pltpu.sync_copy(x_hbm.at[pl.ds(x0, CHUNK * C)], xbuf)
        a0 = pl.multiple_of((g0 // 8) * 8, 8)
        off = g0 - a0
        pltpu.sync_copy(batch_hbm.at[pl.ds(a0, BPAD)], bbuf)

        def row_body(i, rc):
            seg = bbuf[pl.ds(off + i, 16)][0]
            bs = seg * C
            xb = i * C
            for j in range(16):
                xv = xbuf[pl.ds(xb + j * 16, 16)]
                plsc.addupdate(accs.at[pl.ds(bs + j * 16, 16)], xv)
            plsc.addupdate(accc.at[pl.ds(seg * 16, 16)], ones)
            return rc

        lax.fori_loop(0, CHUNK, row_body, 0)
        return carry

    lax.fori_loop(0, NCHUNK, chunk_body, 0)
    pltpu.sync_copy(accs, sums_out.at[wid])
    pltpu.sync_copy(accc, cnt_out.at[wid])


def _mlp_body(sp_ref, cp_ref, w1_ref, w2_ref, scale_ref, offs_ref):
    sums = jnp.sum(sp_ref[...], axis=0)
    cnt = jnp.sum(cp_ref[...], axis=0)
    counts = cnt[:, 0:1]
    inv = 1.0 / jnp.maximum(counts, 1.0)
    mean = sums * inv
    h = lax.dot_general(mean, w1_ref[...], (((1,), (1,)), ((), ())),
                        preferred_element_type=f32)
    h = jnp.maximum(h, 0.0)
    logits = lax.dot_general(h, w2_ref[...], (((1,), (1,)), ((), ())),
                             preferred_element_type=f32)
    scale_ref[...] = 1.0 / (1.0 + jnp.exp(-logits))
    ii = lax.broadcasted_iota(i32, (G, 128), 0)
    jj = lax.broadcasted_iota(i32, (G, 128), 1)
    contrib = jnp.where(ii < jj, counts, 0.0)
    offs = jnp.sum(contrib, axis=0, keepdims=True)
    offs_ref[...] = jnp.broadcast_to(offs, (8, 128)).astype(i32)


def _mlp(sums_p, cnt_p, W1, W2):
    return pl.pallas_call(
        _mlp_body,
        out_shape=(
            jax.ShapeDtypeStruct((G, C), f32),
            jax.ShapeDtypeStruct((8, 128), i32),
        ),
    )(sums_p, cnt_p, W1, W2)


@functools.partial(
    pl.kernel,
    out_type=jax.ShapeDtypeStruct((N * C,), f32),
    mesh=_mesh,
    scratch_types=[
        pltpu.VMEM((CHUNK * C,), f32),
        pltpu.VMEM((G * C,), f32),
        pltpu.VMEM((128,), i32),
    ],
)
def _scale_mul(x_hbm, scale_hbm, offs_hbm, out_hbm, buf, sc_v, offs_v):
    wid = lax.axis_index("s") * 2 + lax.axis_index("c")
    pltpu.sync_copy(scale_hbm, sc_v)
    pltpu.sync_copy(offs_hbm.at[0], offs_v)
    row0 = wid * RPW

    def chunk_body(cix, carry):
        r0 = row0 + cix * CHUNK
        x0 = pl.multiple_of(r0 * C, 8)
        pltpu.sync_copy(x_hbm.at[pl.ds(x0, CHUNK * C)], buf)

        def seg_body(g, sc):
            ov = offs_v[pl.ds(g, 16)]
            lo = jnp.maximum(ov[0], r0)
            hi = jnp.minimum(ov[1], r0 + CHUNK)

            @pl.when(lo < hi)
            def _():
                sb = g * C
                svs = [sc_v[pl.ds(sb + j * 16, 16)] for j in range(16)]

                def row_body(i, rc):
                    b = i * C
                    for j in range(16):
                        s = pl.ds(b + j * 16, 16)
                        buf[s] = buf[s] * svs[j]
                    return rc

                lax.fori_loop(lo - r0, hi - r0, row_body, 0)

            return sc

        lax.fori_loop(0, G, seg_body, 0)
        pltpu.sync_copy(buf, out_hbm.at[pl.ds(x0, CHUNK * C)])
        return carry

    lax.fori_loop(0, NCHUNK, chunk_body, 0)


def kernel(x, batch, W1, W2):
    xf = x.reshape(-1)
    b32 = batch.astype(i32)
    bpad = jnp.concatenate([b32, jnp.zeros((96,), i32)])
    sums_p, cnt_p = _segsum(xf, bpad)
    scale, offs = _mlp(sums_p.reshape(NW, G, C), cnt_p.reshape(NW, G, 16), W1, W2)
    out = _scale_mul(xf, scale.reshape(-1), offs)
    return out.reshape(N, C)
